# baseline (speedup 1.0000x reference)
import jax
import jax.numpy as jnp
from jax import lax
from jax.experimental import pallas as pl
from jax.experimental.pallas import tpu as pltpu
from jax.experimental.pallas import tpu_sc as plsc

_NC = 2
_NS = 16
_NW = _NC * _NS
_D = 64
_TC = 10
_BPW = 32
_Q = 4
_QROWS = _BPW * _TC // _Q


def _build(B, T):
    nchunk = T // _TC
    qtot = nchunk * _Q
    rows = _BPW * _TC
    mesh = plsc.VectorSubcoreMesh(
        core_axis_name="c", subcore_axis_name="s",
        num_cores=_NC, num_subcores=_NS)

    def body(idx_hbm, tok_hbm, pos_hbm, out_hbm,
             idx_v, pos_v, g0, g1, o0, o1, gs0, gs1, os0, os1):
        w = lax.axis_index("s") * _NC + lax.axis_index("c")
        b0 = w * _BPW
        pltpu.sync_copy(idx_hbm.at[w], idx_v)
        pltpu.sync_copy(pos_hbm, pos_v)

        gbuf = (g0, g1)
        obuf = (o0, o1)
        gsem = (gs0, gs1)
        osem = (os0, os1)

        def start_gather(ct, h):
            for q in range(_Q):
                pltpu.async_copy(
                    tok_hbm.at[idx_v.at[ct * _Q + q]],
                    gbuf[h].at[pl.ds(q * _QROWS, _QROWS)], gsem[h])

        def wait_gather(ct, h):
            for q in range(_Q):
                pltpu.make_async_copy(
                    tok_hbm.at[idx_v.at[ct * _Q + q]],
                    gbuf[h].at[pl.ds(q * _QROWS, _QROWS)], gsem[h]).wait()

        def start_out(ct, h):
            pltpu.async_copy(
                obuf[h],
                out_hbm.at[pl.ds(ct * _TC, _TC), slice(None), pl.ds(b0, _BPW)],
                osem[h])

        def wait_out(h):
            pltpu.make_async_copy(
                obuf[h],
                out_hbm.at[pl.ds(0, _TC), slice(None), pl.ds(b0, _BPW)],
                osem[h]).wait()

        lanes = lax.iota(jnp.int32, 16)
        colv = [jnp.full((16,), d, jnp.int32) for d in range(_D)]

        def compute(ct, h):
            src = gbuf[h]
            dst = obuf[h]
            t0 = ct * _TC
            @plsc.parallel_loop(0, _BPW * _TC, unroll=4)
            def badd(row):
                tt = lax.rem(row, _TC)
                for k in range(_D // 16):
                    sl = pl.ds(k * 16, 16)
                    src[row, sl] = src[row, sl] + plsc.load_gather(
                        pos_v, [jnp.full((16,), t0, jnp.int32) + tt,
                                lanes + k * 16])

            for tt in range(_TC):
                r0 = lanes * _TC + tt
                r1 = r0 + 16 * _TC
                for d in range(_D):
                    dst[tt, d, pl.ds(0, 16)] = plsc.load_gather(
                        src, [r0, colv[d]])
                    dst[tt, d, pl.ds(16, 16)] = plsc.load_gather(
                        src, [r1, colv[d]])

        start_gather(0, 0)
        start_gather(1, 1)

        def step(ct, carry):
            def run(h):
                wait_gather(ct, h)

                @pl.when(ct >= 2)
                def _():
                    wait_out(h)

                compute(ct, h)

                @pl.when(ct + 2 < nchunk)
                def _():
                    start_gather(ct + 2, h)

                start_out(ct, h)

            @pl.when(lax.rem(ct, 2) == 0)
            def _():
                run(0)

            @pl.when(lax.rem(ct, 2) == 1)
            def _():
                run(1)

            return carry

        lax.fori_loop(0, nchunk, step, 0)
        wait_out(0)
        wait_out(1)

    return pl.kernel(
        body,
        out_type=jax.ShapeDtypeStruct((T, _D, B), jnp.float32),
        mesh=mesh,
        compiler_params=pltpu.CompilerParams(
            use_tc_tiling_on_sc=False, needs_layout_passes=False),
        scratch_types=[
            pltpu.VMEM((qtot, _QROWS), jnp.int32),
            pltpu.VMEM((T, _D), jnp.float32),
            pltpu.VMEM((rows, _D), jnp.float32),
            pltpu.VMEM((rows, _D), jnp.float32),
            pltpu.VMEM((_TC, _D, _BPW), jnp.float32),
            pltpu.VMEM((_TC, _D, _BPW), jnp.float32),
            pltpu.SemaphoreType.DMA,
            pltpu.SemaphoreType.DMA,
            pltpu.SemaphoreType.DMA,
            pltpu.SemaphoreType.DMA,
        ],
    )


_CB = 128


def _build_conv(V, D):
    nblk = V // _CB
    tail = V % _CB
    per_w = (nblk + _NW - 1) // _NW
    mesh = plsc.VectorSubcoreMesh(
        core_axis_name="c", subcore_axis_name="s",
        num_cores=_NC, num_subcores=_NS)

    def body(tt_hbm, tail_hbm, out_hbm,
             i0, i1, o0, o1, tl, is0, is1, os0, os1):
        w = lax.axis_index("s") * _NC + lax.axis_index("c")
        c_lo = w * per_w
        n = jnp.minimum(c_lo + per_w, nblk) - c_lo
        ibuf = (i0, i1)
        obuf = (o0, o1)
        isem = (is0, is1)
        osem = (os0, os1)

        def start_in(c, h):
            pltpu.async_copy(
                tt_hbm.at[slice(None), pl.ds((c_lo + c) * _CB, _CB)],
                ibuf[h], isem[h])

        def wait_in(h):
            pltpu.make_async_copy(
                tt_hbm.at[slice(None), pl.ds(0, _CB)],
                ibuf[h], isem[h]).wait()

        def start_out(c, h):
            pltpu.async_copy(
                obuf[h],
                out_hbm.at[pl.ds((c_lo + c) * _CB, _CB)],
                osem[h])

        def wait_out(h):
            pltpu.make_async_copy(
                obuf[h],
                out_hbm.at[pl.ds(0, _CB)], osem[h]).wait()

        lanes = lax.iota(jnp.int32, 16)
        rowv = [lanes + k * 16 for k in range(_D // 16)]
        colv = [jnp.full((16,), v, jnp.int32) for v in range(_CB)]

        def compute(h):
            src = ibuf[h]
            dst = obuf[h]
            for v in range(_CB):
                for k in range(_D // 16):
                    dst[v, pl.ds(k * 16, 16)] = plsc.load_gather(
                        src, [rowv[k], colv[v]])

        start_in(0, 0)

        @pl.when(n >= 2)
        def _():
            start_in(1, 1)

        def step(c, carry):
            def run(h):
                wait_in(h)

                @pl.when(c >= 2)
                def _():
                    wait_out(h)

                compute(h)

                @pl.when(c + 2 < n)
                def _():
                    start_in(c + 2, h)

                start_out(c, h)

            @pl.when(lax.rem(c, 2) == 0)
            def _():
                run(0)

            @pl.when(lax.rem(c, 2) == 1)
            def _():
                run(1)

            return carry

        lax.fori_loop(0, n, step, 0)

        @pl.when(n >= 2)
        def _():
            wait_out(0)
            wait_out(1)

        @pl.when(n == 1)
        def _():
            wait_out(0)

        @pl.when(w == _NW - 1)
        def _():
            pltpu.sync_copy(tail_hbm, tl)
            pltpu.sync_copy(tl, out_hbm.at[pl.ds(nblk * _CB, tail)])

    return pl.kernel(
        body,
        out_type=jax.ShapeDtypeStruct((V, 2 * D), jnp.float32),
        mesh=mesh,
        compiler_params=pltpu.CompilerParams(
            use_tc_tiling_on_sc=True, needs_layout_passes=False),
        scratch_types=[
            pltpu.VMEM((_D, _CB), jnp.float32),
            pltpu.VMEM((_D, _CB), jnp.float32),
            pltpu.VMEM((_CB, 2 * _D), jnp.float32),
            pltpu.VMEM((_CB, 2 * _D), jnp.float32),
            pltpu.VMEM((V % _CB, 2 * _D), jnp.float32),
            pltpu.SemaphoreType.DMA,
            pltpu.SemaphoreType.DMA,
            pltpu.SemaphoreType.DMA,
            pltpu.SemaphoreType.DMA,
        ],
    )


def kernel(x, token_table, pos_table):
    B, T = x.shape
    V, D = token_table.shape
    tail = jnp.pad(token_table[(V // _CB) * _CB:, :],
                   ((0, 0), (0, _CB - D)))
    table_wide = _build_conv(V, D)(token_table.T, tail)
    table_lin = table_wide.reshape(2 * V, D)
    xr = (x.astype(jnp.int32)
          .reshape(_NW, _BPW, T // _TC, _TC)
          .transpose(0, 2, 1, 3)
          .reshape(_NW, T // _TC * _Q, _QROWS)) * 2
    out = _build(B, T)(xr, table_lin, pos_table)
    return out.transpose(2, 0, 1)


# speedup vs baseline: 2.2716x; 2.2716x over previous
import jax
import jax.numpy as jnp
from jax import lax
from jax.experimental import pallas as pl
from jax.experimental.pallas import tpu as pltpu
from jax.experimental.pallas import tpu_sc as plsc

_NC = 2
_NS = 16
_NW = _NC * _NS
_D = 64
_TC = 10
_BPW = 32
_Q = 4
_QROWS = _BPW * _TC // _Q


def _build(B, T):
    nchunk = T // _TC
    qtot = nchunk * _Q
    rows = _BPW * _TC
    mesh = plsc.VectorSubcoreMesh(
        core_axis_name="c", subcore_axis_name="s",
        num_cores=_NC, num_subcores=_NS)

    def body(idx_hbm, tok_hbm, pos_hbm, out_hbm,
             idx_v, pos_v, g0, g1, o0, o1, gs0, gs1, os0, os1):
        w = lax.axis_index("s") * _NC + lax.axis_index("c")
        b0 = w * _BPW
        pltpu.sync_copy(idx_hbm.at[w], idx_v)
        pltpu.sync_copy(pos_hbm, pos_v)

        gbuf = (g0, g1)
        obuf = (o0, o1)
        gsem = (gs0, gs1)
        osem = (os0, os1)

        def start_gather(ct, h):
            for q in range(_Q):
                pltpu.async_copy(
                    tok_hbm.at[idx_v.at[ct * _Q + q]],
                    gbuf[h].at[pl.ds(q * _QROWS, _QROWS)], gsem[h])

        def wait_gather(ct, h):
            for q in range(_Q):
                pltpu.make_async_copy(
                    tok_hbm.at[idx_v.at[ct * _Q + q]],
                    gbuf[h].at[pl.ds(q * _QROWS, _QROWS)], gsem[h]).wait()

        def start_out(ct, h):
            pltpu.async_copy(
                obuf[h],
                out_hbm.at[pl.ds(ct * _TC, _TC), slice(None), pl.ds(b0, _BPW)],
                osem[h])

        def wait_out(h):
            pltpu.make_async_copy(
                obuf[h],
                out_hbm.at[pl.ds(0, _TC), slice(None), pl.ds(b0, _BPW)],
                osem[h]).wait()

        def compute(ct, h):
            src = gbuf[h]
            dst = obuf[h]
            t0 = ct * _TC
            lanes = lax.iota(jnp.int32, 16)
            for tt in range(_TC):
                pvs = [pos_v[t0 + tt, pl.ds(k * 16, 16)]
                       for k in range(_D // 16)]

                @plsc.parallel_loop(0, _BPW, unroll=4)
                def badd(b):
                    row = b * _TC + tt
                    for k in range(_D // 16):
                        sl = pl.ds(k * 16, 16)
                        src[row, sl] = src[row, sl] + pvs[k]

                r0 = lanes * _TC + tt
                r1 = r0 + 16 * _TC

                @plsc.parallel_loop(0, _D, unroll=4)
                def dloop(d):
                    cols = lax.broadcast_in_dim(d, (16,), ())
                    dst[tt, d, pl.ds(0, 16)] = plsc.load_gather(src, [r0, cols])
                    dst[tt, d, pl.ds(16, 16)] = plsc.load_gather(src, [r1, cols])

        start_gather(0, 0)
        start_gather(1, 1)

        def step(g, carry):
            for h in range(2):
                ct = 2 * g + h
                wait_gather(ct, h)

                @pl.when(g >= 1)
                def _():
                    wait_out(h)

                compute(ct, h)

                @pl.when(g < nchunk // 2 - 1)
                def _():
                    start_gather(ct + 2, h)

                start_out(ct, h)
            return carry

        lax.fori_loop(0, nchunk // 2, step, 0)
        wait_out(0)
        wait_out(1)

    return pl.kernel(
        body,
        out_type=jax.ShapeDtypeStruct((T, _D, B), jnp.float32),
        mesh=mesh,
        compiler_params=pltpu.CompilerParams(
            use_tc_tiling_on_sc=False, needs_layout_passes=False),
        scratch_types=[
            pltpu.VMEM((qtot, _QROWS), jnp.int32),
            pltpu.VMEM((T, _D), jnp.float32),
            pltpu.VMEM((rows, _D), jnp.float32),
            pltpu.VMEM((rows, _D), jnp.float32),
            pltpu.VMEM((_TC, _D, _BPW), jnp.float32),
            pltpu.VMEM((_TC, _D, _BPW), jnp.float32),
            pltpu.SemaphoreType.DMA,
            pltpu.SemaphoreType.DMA,
            pltpu.SemaphoreType.DMA,
            pltpu.SemaphoreType.DMA,
        ],
    )


def kernel(x, token_table, pos_table):
    B, T = x.shape
    _, D = token_table.shape
    xr = (x.astype(jnp.int32)
          .reshape(_NW, _BPW, T // _TC, _TC)
          .transpose(0, 2, 1, 3)
          .reshape(_NW, T // _TC * _Q, _QROWS))
    out = _build(B, T)(xr, token_table, pos_table)
    return out.transpose(2, 0, 1)
